# baseline (speedup 1.0000x reference)
import functools

import jax
import jax.numpy as jnp
from jax import lax
from jax.experimental import pallas as pl
from jax.experimental.pallas import tpu as pltpu
from jax.experimental.pallas import tpu_sc as plsc

_VOCAB = 100000
_DIM = 64
_B = 16384
_K = 5

_NC = 2
_NS = 16
_NW = _NC * _NS
_BPW = _B // _NW
_S = 128
_NCHUNK = _BPW // _S
_G = _S // 16


@functools.cache
def _make_sc_logits():
    mesh = plsc.VectorSubcoreMesh(core_axis_name="c", subcore_axis_name="s")

    @functools.partial(
        pl.kernel,
        mesh=mesh,
        compiler_params=pltpu.CompilerParams(
            needs_layout_passes=False, use_tc_tiling_on_sc=True),
        out_type=jax.ShapeDtypeStruct((6 * _B,), jnp.float32),
        scratch_types=[
            pltpu.VMEM((_S,), jnp.int32),
            pltpu.VMEM((_S,), jnp.int32),
            pltpu.VMEM((_K, _S), jnp.int32),
            pltpu.VMEM((_S,), jnp.int32),
            pltpu.VMEM((_S,), jnp.int32),
            pltpu.VMEM((_K, _S), jnp.int32),
            pltpu.VMEM((_S, 2 * _DIM), jnp.float32),
            pltpu.VMEM((_S, 2 * _DIM), jnp.float32),
            pltpu.VMEM((_K, _S, 2 * _DIM), jnp.float32),
            pltpu.VMEM((6, _S), jnp.float32),
            pltpu.SemaphoreType.DMA,
            pltpu.SemaphoreType.DMA,
            pltpu.SemaphoreType.DMA,
        ],
    )
    def sc_logits(gc_hbm, gx_hbm, gn_hbm, pc_hbm, px_hbm, pn_hbm,
                  win_hbm, wout_hbm, out_hbm,
                  idxc, idxx, idxn, parc, parx, parn,
                  crows, xrows, nrows, lbuf, semc, semx, semn):
        wid = lax.axis_index("s") * _NC + lax.axis_index("c")
        base = wid * _BPW
        iota = lax.iota(jnp.int32, 16)

        def chunk_body(t, carry):
            cbase = pl.multiple_of(base + t * _S, _S)
            pltpu.sync_copy(gc_hbm.at[pl.ds(cbase, _S)], idxc)
            pltpu.sync_copy(pc_hbm.at[pl.ds(cbase, _S)], parc)
            pltpu.sync_copy(gx_hbm.at[pl.ds(cbase, _S)], idxx)
            pltpu.sync_copy(px_hbm.at[pl.ds(cbase, _S)], parx)
            for j in range(_K):
                nbase = pl.multiple_of(j * _B + cbase, _S)
                pltpu.sync_copy(gn_hbm.at[pl.ds(nbase, _S)], idxn.at[j])
                pltpu.sync_copy(pn_hbm.at[pl.ds(nbase, _S)], parn.at[j])
            cps = [pltpu.async_copy(win_hbm.at[idxc], crows, semc),
                   pltpu.async_copy(wout_hbm.at[idxx], xrows, semx)]
            cps += [pltpu.async_copy(wout_hbm.at[idxn.at[j]], nrows.at[j], semn)
                    for j in range(_K)]
            for cp in cps:
                cp.wait()

            def g_body(g, carry2):
                s0 = pl.multiple_of(g * 16, 16)
                pcv = parc[pl.ds(s0, 16)] * 64
                pxv = parx[pl.ds(s0, 16)] * 64
                pnv = [parn[j, pl.ds(s0, 16)] * 64 for j in range(_K)]
                accs = [jnp.zeros((16,), jnp.float32) for _ in range(6)]
                for l in range(16):
                    s = s0 + l
                    lane = iota == l
                    oc = pl.multiple_of(pcv[l], 64)
                    cvs = [crows[s, pl.ds(oc + k * 16, 16)]
                           for k in range(_DIM // 16)]
                    for j in range(6):
                        if j == 0:
                            ox = pl.multiple_of(pxv[l], 64)
                            rvs = [xrows[s, pl.ds(ox + k * 16, 16)]
                                   for k in range(_DIM // 16)]
                        else:
                            on = pl.multiple_of(pnv[j - 1][l], 64)
                            rvs = [nrows[j - 1, s, pl.ds(on + k * 16, 16)]
                                   for k in range(_DIM // 16)]
                        p = cvs[0] * rvs[0]
                        for k in range(1, _DIM // 16):
                            p = p + cvs[k] * rvs[k]
                        r = jnp.sum(p)
                        accs[j] = jnp.where(lane, r, accs[j])
                for j in range(6):
                    lbuf[j, pl.ds(s0, 16)] = accs[j]
                return carry2

            lax.fori_loop(0, _G, g_body, 0)
            for j in range(6):
                obase = pl.multiple_of(j * _B + cbase, 128)
                pltpu.sync_copy(lbuf.at[j], out_hbm.at[pl.ds(obase, _S)])
            return carry

        lax.fori_loop(0, _NCHUNK, chunk_body, 0)

    return sc_logits


def _bce_body(x_ref, o_ref):
    x = x_ref[...]
    pos_rows = _B // 128
    lbl = (lax.broadcasted_iota(jnp.int32, x.shape, 0) < pos_rows
           ).astype(jnp.float32)
    v = jnp.maximum(x, 0.0) - x * lbl + jnp.log(1.0 + jnp.exp(-jnp.abs(x)))
    o_ref[0, 0] = jnp.sum(v) / (6.0 * _B)


def kernel(center, context, negatives, W_in, W_out):
    cen = center.astype(jnp.int32)
    ctx = context.reshape(_B).astype(jnp.int32)
    neg_t = negatives.astype(jnp.int32).T.reshape(_K * _B)
    wi2 = jnp.concatenate([W_in[0::2], W_in[1::2]], axis=1)
    wo2 = jnp.concatenate([W_out[0::2], W_out[1::2]], axis=1)
    logits = _make_sc_logits()(
        cen >> 1, ctx >> 1, neg_t >> 1,
        cen & 1, ctx & 1, neg_t & 1,
        wi2, wo2)
    loss = pl.pallas_call(
        _bce_body,
        out_shape=jax.ShapeDtypeStruct((1, 1), jnp.float32),
        out_specs=pl.BlockSpec(memory_space=pltpu.SMEM),
    )(logits.reshape(6 * _B // 128, 128))
    return loss[0, 0]


# speedup vs baseline: 8.9753x; 8.9753x over previous
import functools

import jax
import jax.numpy as jnp
from jax import lax
from jax.experimental import pallas as pl
from jax.experimental.pallas import tpu as pltpu
from jax.experimental.pallas import tpu_sc as plsc

_VOCAB = 100000
_DIM = 64
_B = 16384
_K = 5

_NC = 2
_NS = 16
_NW = _NC * _NS
_BPW = _B // _NW
_S = 128
_NCHUNK = _BPW // _S
_G = _S // 16


@functools.cache
def _make_sc_logits():
    mesh = plsc.VectorSubcoreMesh(core_axis_name="c", subcore_axis_name="s")

    @functools.partial(
        pl.kernel,
        mesh=mesh,
        compiler_params=pltpu.CompilerParams(
            needs_layout_passes=False, use_tc_tiling_on_sc=True),
        out_type=jax.ShapeDtypeStruct((6 * _B,), jnp.float32),
        scratch_types=[
            pltpu.VMEM((_S,), jnp.int32),
            pltpu.VMEM((_S,), jnp.int32),
            pltpu.VMEM((_K, _S), jnp.int32),
            pltpu.VMEM((_S,), jnp.int32),
            pltpu.VMEM((_S,), jnp.int32),
            pltpu.VMEM((_K, _S), jnp.int32),
            pltpu.VMEM((_S, 2 * _DIM), jnp.float32),
            pltpu.VMEM((_S, 2 * _DIM), jnp.float32),
            pltpu.VMEM((_K, _S, 2 * _DIM), jnp.float32),
            pltpu.VMEM((6, _S), jnp.float32),
            pltpu.SemaphoreType.DMA,
            pltpu.SemaphoreType.DMA,
            pltpu.SemaphoreType.DMA,
        ],
    )
    def sc_logits(gc_hbm, gx_hbm, gn_hbm, pc_hbm, px_hbm, pn_hbm,
                  win_hbm, wout_hbm, out_hbm,
                  idxc, idxx, idxn, parc, parx, parn,
                  crows, xrows, nrows, lbuf, semc, semx, semn):
        wid = lax.axis_index("s") * _NC + lax.axis_index("c")
        base = wid * _BPW
        iota = lax.iota(jnp.int32, 16)

        def chunk_body(t, carry):
            cbase = pl.multiple_of(base + t * _S, _S)
            pltpu.sync_copy(gc_hbm.at[pl.ds(cbase, _S)], idxc)
            pltpu.sync_copy(pc_hbm.at[pl.ds(cbase, _S)], parc)
            pltpu.sync_copy(gx_hbm.at[pl.ds(cbase, _S)], idxx)
            pltpu.sync_copy(px_hbm.at[pl.ds(cbase, _S)], parx)
            for j in range(_K):
                nbase = pl.multiple_of(j * _B + cbase, _S)
                pltpu.sync_copy(gn_hbm.at[pl.ds(nbase, _S)], idxn.at[j])
                pltpu.sync_copy(pn_hbm.at[pl.ds(nbase, _S)], parn.at[j])
            cps = [pltpu.async_copy(win_hbm.at[idxc], crows, semc),
                   pltpu.async_copy(wout_hbm.at[idxx], xrows, semx)]
            cps += [pltpu.async_copy(wout_hbm.at[idxn.at[j]], nrows.at[j], semn)
                    for j in range(_K)]
            for cp in cps:
                cp.wait()

            def g_body(g, carry2):
                s0 = pl.multiple_of(g * 16, 16)
                pcv = parc[pl.ds(s0, 16)] * 64
                pxv = parx[pl.ds(s0, 16)] * 64
                pnv = [parn[j, pl.ds(s0, 16)] * 64 for j in range(_K)]
                accs = [jnp.zeros((16,), jnp.float32) for _ in range(6)]
                for l in range(16):
                    s = s0 + l
                    lane = iota == l
                    oc = pl.multiple_of(pcv[l], 64)
                    cvs = [crows[s, pl.ds(oc + k * 16, 16)]
                           for k in range(_DIM // 16)]
                    for j in range(6):
                        if j == 0:
                            ox = pl.multiple_of(pxv[l], 64)
                            rvs = [xrows[s, pl.ds(ox + k * 16, 16)]
                                   for k in range(_DIM // 16)]
                        else:
                            on = pl.multiple_of(pnv[j - 1][l], 64)
                            rvs = [nrows[j - 1, s, pl.ds(on + k * 16, 16)]
                                   for k in range(_DIM // 16)]
                        p = cvs[0] * rvs[0]
                        for k in range(1, _DIM // 16):
                            p = p + cvs[k] * rvs[k]
                        r = jnp.sum(p)
                        accs[j] = jnp.where(lane, r, accs[j])
                for j in range(6):
                    lbuf[j, pl.ds(s0, 16)] = accs[j]
                return carry2

            lax.fori_loop(0, _G, g_body, 0)
            for j in range(6):
                obase = pl.multiple_of(j * _B + cbase, 128)
                pltpu.sync_copy(lbuf.at[j], out_hbm.at[pl.ds(obase, _S)])
            return carry

        lax.fori_loop(0, _NCHUNK, chunk_body, 0)

    return sc_logits


def _bce_body(x_ref, o_ref):
    x = x_ref[...]
    pos_rows = _B // 128
    lbl = (lax.broadcasted_iota(jnp.int32, x.shape, 0) < pos_rows
           ).astype(jnp.float32)
    v = jnp.maximum(x, 0.0) - x * lbl + jnp.log(1.0 + jnp.exp(-jnp.abs(x)))
    o_ref[0, 0] = jnp.sum(v) / (6.0 * _B)


def kernel(center, context, negatives, W_in, W_out):
    cen = center.astype(jnp.int32)
    ctx = context.reshape(_B).astype(jnp.int32)
    neg_t = negatives.astype(jnp.int32).T.reshape(_K * _B)
    one = (cen[0] & 0).astype(jnp.float32) + 1.0
    wi2 = (W_in * one).reshape(_VOCAB // 2, 2 * _DIM)
    wo2 = (W_out * one).reshape(_VOCAB // 2, 2 * _DIM)
    logits = _make_sc_logits()(
        cen >> 1, ctx >> 1, neg_t >> 1,
        cen & 1, ctx & 1, neg_t & 1,
        wi2, wo2)
    loss = pl.pallas_call(
        _bce_body,
        out_shape=jax.ShapeDtypeStruct((1, 1), jnp.float32),
        out_specs=pl.BlockSpec(memory_space=pltpu.SMEM),
    )(logits.reshape(6 * _B // 128, 128))
    return loss[0, 0]


# speedup vs baseline: 11.7896x; 1.3136x over previous
import functools

import jax
import jax.numpy as jnp
from jax import lax
from jax.experimental import pallas as pl
from jax.experimental.pallas import tpu as pltpu
from jax.experimental.pallas import tpu_sc as plsc

_VOCAB = 100000
_DIM = 64
_B = 16384
_K = 5

_NC = 2
_NS = 16
_NW = _NC * _NS
_BPW = _B // _NW
_S = 64
_NCHUNK = _BPW // _S
_G = _S // 16


@functools.cache
def _make_sc_logits():
    mesh = plsc.VectorSubcoreMesh(core_axis_name="c", subcore_axis_name="s")

    @functools.partial(
        pl.kernel,
        mesh=mesh,
        compiler_params=pltpu.CompilerParams(
            needs_layout_passes=False, use_tc_tiling_on_sc=False),
        out_type=jax.ShapeDtypeStruct((6 * _B,), jnp.float32),
        scratch_types=[
            pltpu.VMEM((_BPW,), jnp.int32),
            pltpu.VMEM((_BPW,), jnp.int32),
            pltpu.VMEM((_K * _BPW,), jnp.int32),
            pltpu.VMEM((_S, _DIM), jnp.float32),
            pltpu.VMEM((_S, _DIM), jnp.float32),
            pltpu.VMEM((_K * _S, _DIM), jnp.float32),
            pltpu.VMEM((_S, _DIM), jnp.float32),
            pltpu.VMEM((_S, _DIM), jnp.float32),
            pltpu.VMEM((_K * _S, _DIM), jnp.float32),
            pltpu.VMEM((6, _BPW), jnp.float32),
            pltpu.SemaphoreType.DMA,
            pltpu.SemaphoreType.DMA,
            pltpu.SemaphoreType.DMA,
        ],
    )
    def sc_logits(cen_hbm, ctx_hbm, neg_hbm, win_hbm, wout_hbm, out_hbm,
                  idxc, idxx, idxn,
                  crA, xrA, nrA, crB, xrB, nrB,
                  lbuf, semi, semA, semB):
        wid = lax.axis_index("s") * _NC + lax.axis_index("c")
        base = wid * _BPW
        iota = lax.iota(jnp.int32, 16)

        cpi = [pltpu.async_copy(cen_hbm.at[pl.ds(base, _BPW)], idxc, semi),
               pltpu.async_copy(ctx_hbm.at[pl.ds(base, _BPW)], idxx, semi),
               pltpu.async_copy(neg_hbm.at[pl.ds(base * _K, _K * _BPW)],
                                idxn, semi)]
        for cp in cpi:
            cp.wait()

        def gather_bufs(t, cr, xr, nr, sem):
            toff = pl.multiple_of(t * _S, _S)
            return [
                pltpu.async_copy(win_hbm.at[idxc.at[pl.ds(toff, _S)]],
                                 cr, sem),
                pltpu.async_copy(wout_hbm.at[idxx.at[pl.ds(toff, _S)]],
                                 xr, sem),
                pltpu.async_copy(wout_hbm.at[idxn.at[pl.ds(toff * _K,
                                                           _K * _S)]],
                                 nr, sem),
            ]

        def wait_bufs(t, cr, xr, nr, sem):
            toff = pl.multiple_of(t * _S, _S)
            pltpu.make_async_copy(win_hbm.at[idxc.at[pl.ds(toff, _S)]],
                                  cr, sem).wait()
            pltpu.make_async_copy(wout_hbm.at[idxx.at[pl.ds(toff, _S)]],
                                  xr, sem).wait()
            pltpu.make_async_copy(wout_hbm.at[idxn.at[pl.ds(toff * _K,
                                                            _K * _S)]],
                                  nr, sem).wait()

        def compute_chunk(t, cr, xr, nr):
            toff = pl.multiple_of(t * _S, _S)

            def g_body(g, carry):
                s0 = pl.multiple_of(g * 16, 16)
                accs = [jnp.zeros((16,), jnp.float32) for _ in range(6)]
                for l in range(16):
                    s = s0 + l
                    lane = iota == l
                    cvs = [cr[s, pl.ds(k * 16, 16)]
                           for k in range(_DIM // 16)]
                    for j in range(6):
                        if j == 0:
                            rvs = [xr[s, pl.ds(k * 16, 16)]
                                   for k in range(_DIM // 16)]
                        else:
                            rvs = [nr[s * _K + (j - 1), pl.ds(k * 16, 16)]
                                   for k in range(_DIM // 16)]
                        p = cvs[0] * rvs[0]
                        for k in range(1, _DIM // 16):
                            p = p + cvs[k] * rvs[k]
                        r = jnp.sum(p)
                        accs[j] = jnp.where(lane, r, accs[j])
                for j in range(6):
                    lbuf[j, pl.ds(toff + s0, 16)] = accs[j]
                return carry

            lax.fori_loop(0, _G, g_body, 0)

        gather_bufs(0, crA, xrA, nrA, semA)

        def pair_body(pr, carry):
            t0 = pr * 2
            t1 = t0 + 1
            gather_bufs(t1, crB, xrB, nrB, semB)
            wait_bufs(t0, crA, xrA, nrA, semA)
            compute_chunk(t0, crA, xrA, nrA)

            @pl.when(pr < _NCHUNK // 2 - 1)
            def _():
                gather_bufs(t0 + 2, crA, xrA, nrA, semA)

            wait_bufs(t1, crB, xrB, nrB, semB)
            compute_chunk(t1, crB, xrB, nrB)
            return carry

        lax.fori_loop(0, _NCHUNK // 2, pair_body, 0)

        for j in range(6):
            obase = pl.multiple_of(j * _B + base, _BPW)
            pltpu.sync_copy(lbuf.at[j], out_hbm.at[pl.ds(obase, _BPW)])

    return sc_logits


def _bce_body(x_ref, o_ref):
    x = x_ref[...]
    pos_rows = _B // 128
    lbl = (lax.broadcasted_iota(jnp.int32, x.shape, 0) < pos_rows
           ).astype(jnp.float32)
    v = jnp.maximum(x, 0.0) - x * lbl + jnp.log(1.0 + jnp.exp(-jnp.abs(x)))
    o_ref[0, 0] = jnp.sum(v) / (6.0 * _B)


def kernel(center, context, negatives, W_in, W_out):
    cen = center.astype(jnp.int32)
    ctx = context.reshape(_B).astype(jnp.int32)
    neg = negatives.reshape(_B * _K).astype(jnp.int32)
    logits = _make_sc_logits()(cen, ctx, neg, W_in, W_out)
    loss = pl.pallas_call(
        _bce_body,
        out_shape=jax.ShapeDtypeStruct((1, 1), jnp.float32),
        out_specs=pl.BlockSpec(memory_space=pltpu.SMEM),
    )(logits.reshape(6 * _B // 128, 128))
    return loss[0, 0]
